# Initial kernel scaffold; baseline (speedup 1.0000x reference)
import jax
import jax.numpy as jnp
from jax.experimental import pallas as pl


def kernel(item_embedding, cate_embedding, edge_index, edge_vals):
    raise NotImplementedError("write your pallas kernel here")


# speedup vs baseline: 3.8495x; 3.8495x over previous
import functools

import jax
import jax.numpy as jnp
from jax import lax
from jax.experimental import pallas as pl
from jax.experimental.pallas import tpu as pltpu
from jax.experimental.pallas import tpu_sc as plsc

_EMB = 128
_L = 16
_NC = 2
_NS = 16
_NW = _NC * _NS
_CH = _EMB // _L


def _make_sc_layer(n_pad, n_edges):
    epw = n_edges // _NW
    assert epw * _NW == n_edges
    K = 80
    assert epw % K == 0
    nb = epw // K
    rpt = n_pad // _NS
    assert rpt % 8 == 0
    zr = 96
    assert rpt % zr == 0
    nz = rpt // zr

    mesh = plsc.VectorSubcoreMesh(core_axis_name="c", subcore_axis_name="s")

    @functools.partial(
        pl.kernel,
        mesh=mesh,
        out_type=jax.ShapeDtypeStruct((_NC, n_pad, _EMB), jnp.float32),
        scratch_types=[
            pltpu.VMEM((K,), jnp.int32),
            pltpu.VMEM((K,), jnp.int32),
            pltpu.VMEM((K,), jnp.float32),
            pltpu.VMEM((K, _EMB), jnp.float32),
            pltpu.VMEM((zr, _EMB), jnp.float32),
            pltpu.VMEM_SHARED((n_pad, _EMB), jnp.float32),
            pltpu.SemaphoreType.DMA,
        ],
    )
    def layer(table, src, dst, val, out, src_v, dst_v, val_v, rows_v, zbuf,
              acc, sem):
        cid = lax.axis_index("c")
        sid = lax.axis_index("s")
        wid = sid * _NC + cid
        base = wid * epw
        r0 = sid * rpt

        def zrow(r, carry):
            for c in range(_CH):
                zbuf[r, pl.ds(c * _L, _L)] = jnp.zeros((_L,), jnp.float32)
            return carry

        lax.fori_loop(0, zr, zrow, 0)

        def zcpy(i, carry):
            pltpu.sync_copy(zbuf, acc.at[pl.ds(r0 + i * zr, zr)])
            return carry

        lax.fori_loop(0, nz, zcpy, 0)
        plsc.subcore_barrier()

        def blk(b, carry):
            off = base + b * K
            pltpu.sync_copy(src.at[pl.ds(off, K)], src_v)
            pltpu.sync_copy(dst.at[pl.ds(off, K)], dst_v)
            pltpu.sync_copy(val.at[pl.ds(off, K)], val_v)
            pltpu.async_copy(table.at[src_v], rows_v, sem).wait()

            def scale(eb, inner):
                vv = val_v[pl.ds(eb * _L, _L)]
                for i in range(_L):
                    v = vv[i]
                    e = eb * _L + i
                    for c in range(_CH):
                        sl = pl.ds(c * _L, _L)
                        rows_v[e, sl] = rows_v[e, sl] * v
                return inner

            lax.fori_loop(0, K // _L, scale, 0)
            pltpu.sync_copy(rows_v, acc.at[dst_v], add=True)
            return carry

        lax.fori_loop(0, nb, blk, 0)
        plsc.subcore_barrier()

        pltpu.sync_copy(acc.at[pl.ds(r0, rpt)], out.at[cid, pl.ds(r0, rpt)])

    return layer


def _tc_add(a, b):
    n, d = a.shape
    blk = n // _NS
    assert blk * _NS == n and blk % 8 == 0
    return pl.pallas_call(
        lambda x, y, o: o.__setitem__(..., x[...] + y[...]),
        out_shape=jax.ShapeDtypeStruct((n, d), jnp.float32),
        grid=(_NS,),
        in_specs=[pl.BlockSpec((blk, d), lambda i: (i, 0))] * 2,
        out_specs=pl.BlockSpec((blk, d), lambda i: (i, 0)),
    )(a, b)


def _tc_final(t0, t1, p2a, p2b, n_items):
    d = t0.shape[1]
    blk = n_items // 10
    assert blk * 10 == n_items and blk % 8 == 0

    def body(a, b, c, e, o):
        o[...] = (a[...] + b[...] + c[...] + e[...]) * jnp.float32(1.0 / 3.0)

    return pl.pallas_call(
        body,
        out_shape=jax.ShapeDtypeStruct((n_items, d), jnp.float32),
        grid=(10,),
        in_specs=[pl.BlockSpec((blk, d), lambda i: (i, 0))] * 4,
        out_specs=pl.BlockSpec((blk, d), lambda i: (i, 0)),
    )(t0, t1, p2a, p2b)


def kernel(item_embedding, cate_embedding, edge_index, edge_vals):
    n_items = item_embedding.shape[0]
    t0 = jnp.concatenate([item_embedding, cate_embedding], axis=0)
    n_total = t0.shape[0]
    n_edges = edge_index.shape[1]

    unit = _NS * 96
    n_pad = ((n_total + unit - 1) // unit) * unit
    t0p = jnp.pad(t0, ((0, n_pad - n_total), (0, 0)))

    src = edge_index[0].astype(jnp.int32)
    dst = edge_index[1].astype(jnp.int32)
    val = edge_vals.astype(jnp.float32)

    sc_layer = _make_sc_layer(n_pad, n_edges)
    p1 = sc_layer(t0p, src, dst, val)
    t1 = _tc_add(p1[0], p1[1])
    p2 = sc_layer(t1, src, dst, val)
    return _tc_final(t0p[:n_items], t1[:n_items], p2[0, :n_items],
                     p2[1, :n_items], n_items)


# speedup vs baseline: 4.3943x; 1.1415x over previous
import functools

import jax
import jax.numpy as jnp
from jax import lax
from jax.experimental import pallas as pl
from jax.experimental.pallas import tpu as pltpu
from jax.experimental.pallas import tpu_sc as plsc

_EMB = 128
_L = 16
_NC = 2
_NS = 16
_NW = _NC * _NS
_CH = _EMB // _L


def _make_sc_layer(n_pad, n_edges):
    epw = n_edges // _NW
    assert epw * _NW == n_edges
    K = 80
    assert epw % K == 0 and K % _L == 0
    nb = epw // K
    assert nb % 2 == 1
    npairs = (nb - 1) // 2
    rpt = n_pad // _NS
    assert rpt % 8 == 0
    zr = 32
    assert rpt % zr == 0
    nz = rpt // zr

    mesh = plsc.VectorSubcoreMesh(core_axis_name="c", subcore_axis_name="s")

    @functools.partial(
        pl.kernel,
        mesh=mesh,
        out_type=jax.ShapeDtypeStruct((_NC, n_pad, _EMB), jnp.float32),
        scratch_types=[
            pltpu.VMEM((2 * K,), jnp.int32),
            pltpu.VMEM((2 * K,), jnp.int32),
            pltpu.VMEM((K,), jnp.float32),
            pltpu.VMEM((K,), jnp.float32),
            pltpu.VMEM((K, _EMB), jnp.float32),
            pltpu.VMEM((K, _EMB), jnp.float32),
            pltpu.VMEM((K,), jnp.int32),
            pltpu.VMEM((K,), jnp.int32),
            pltpu.VMEM((K,), jnp.int32),
            pltpu.VMEM((zr, _EMB), jnp.float32),
            pltpu.VMEM_SHARED((n_pad, _EMB), jnp.float32),
            pltpu.SemaphoreType.DMA,
            pltpu.SemaphoreType.DMA,
            pltpu.SemaphoreType.DMA,
            pltpu.SemaphoreType.DMA,
        ],
    )
    def layer(table, rec, valr, out, ebuf_a, ebuf_b, vbuf_a, vbuf_b,
              rows_a, rows_b, sidx, gidx_a, gidx_b, zbuf, acc, esem_a,
              esem_b, gsem_a, gsem_b):
        cid = lax.axis_index("c")
        sid = lax.axis_index("s")
        wid = sid * _NC + cid
        r0 = sid * rpt
        ebase = wid * (nb + 2) * 2 * K
        vbase = wid * (nb + 2) * K

        def zrow(r, carry):
            for c in range(_CH):
                zbuf[r, pl.ds(c * _L, _L)] = jnp.zeros((_L,), jnp.float32)
            return carry

        lax.fori_loop(0, zr, zrow, 0)

        def zcpy(i, carry):
            pltpu.sync_copy(zbuf, acc.at[pl.ds(r0 + i * zr, zr)])
            return carry

        lax.fori_loop(0, nz, zcpy, 0)
        plsc.subcore_barrier()

        def issue_edges(i, ebuf, vbuf, esem):
            pltpu.async_copy(rec.at[pl.ds(ebase + i * 2 * K, 2 * K)], ebuf,
                             esem)
            pltpu.async_copy(valr.at[pl.ds(vbase + i * K, K)], vbuf, esem)

        def wait_edges(ebuf, vbuf, esem):
            pltpu.make_async_copy(rec.at[pl.ds(ebase, 2 * K)], ebuf,
                                  esem).wait()
            pltpu.make_async_copy(valr.at[pl.ds(vbase, K)], vbuf,
                                  esem).wait()

        def issue_gather(ebuf, gidx, rbuf, gsem):
            for c in range(K // _L):
                gidx[pl.ds(c * _L, _L)] = ebuf[pl.ds(c * _L, _L)]
            pltpu.async_copy(table.at[gidx], rbuf, gsem)

        def wait_gather(rbuf, gsem):
            pltpu.make_async_copy(table.at[sidx], rbuf, gsem).wait()

        def scale_scatter(ebuf, vbuf, rbuf):
            def grp(eb, inner):
                vv = vbuf[pl.ds(eb * _L, _L)]
                for i in range(_L):
                    v = vv[i]
                    e = eb * _L + i
                    for c in range(_CH):
                        sl = pl.ds(c * _L, _L)
                        rbuf[e, sl] = rbuf[e, sl] * v
                return inner

            lax.fori_loop(0, K // _L, grp, 0)
            for c in range(K // _L):
                sidx[pl.ds(c * _L, _L)] = ebuf[pl.ds(K + c * _L, _L)]
            pltpu.sync_copy(rbuf, acc.at[sidx], add=True)

        def half(i, s_ebuf, s_vbuf, s_rows, s_esem, s_gsem, o_ebuf, o_vbuf,
                 o_gidx, o_rows, o_esem, o_gsem):
            wait_gather(s_rows, s_gsem)
            scale_scatter(s_ebuf, s_vbuf, s_rows)
            issue_edges(i + 2, s_ebuf, s_vbuf, s_esem)
            wait_edges(o_ebuf, o_vbuf, o_esem)
            issue_gather(o_ebuf, o_gidx, o_rows, o_gsem)

        issue_edges(0, ebuf_a, vbuf_a, esem_a)
        issue_edges(1, ebuf_b, vbuf_b, esem_b)
        wait_edges(ebuf_a, vbuf_a, esem_a)
        issue_gather(ebuf_a, gidx_a, rows_a, gsem_a)

        def pair(p, carry):
            b0 = 2 * p
            half(b0, ebuf_a, vbuf_a, rows_a, esem_a, gsem_a, ebuf_b,
                 vbuf_b, gidx_b, rows_b, esem_b, gsem_b)
            half(b0 + 1, ebuf_b, vbuf_b, rows_b, esem_b, gsem_b, ebuf_a,
                 vbuf_a, gidx_a, rows_a, esem_a, gsem_a)
            return carry

        lax.fori_loop(0, npairs, pair, 0)
        half(nb - 1, ebuf_a, vbuf_a, rows_a, esem_a, gsem_a, ebuf_b,
             vbuf_b, gidx_b, rows_b, esem_b, gsem_b)
        wait_gather(rows_b, gsem_b)
        wait_edges(ebuf_a, vbuf_a, esem_a)

        plsc.subcore_barrier()

        pltpu.sync_copy(acc.at[pl.ds(r0, rpt)], out.at[cid, pl.ds(r0, rpt)])

    return layer


def _tc_add(a, b):
    n, d = a.shape
    blk = n // _NS
    assert blk * _NS == n and blk % 8 == 0
    return pl.pallas_call(
        lambda x, y, o: o.__setitem__(..., x[...] + y[...]),
        out_shape=jax.ShapeDtypeStruct((n, d), jnp.float32),
        grid=(_NS,),
        in_specs=[pl.BlockSpec((blk, d), lambda i: (i, 0))] * 2,
        out_specs=pl.BlockSpec((blk, d), lambda i: (i, 0)),
    )(a, b)


def _tc_final(t0, t1, p2a, p2b, n_items):
    d = t0.shape[1]
    blk = n_items // 10
    assert blk * 10 == n_items and blk % 8 == 0

    def body(a, b, c, e, o):
        o[...] = (a[...] + b[...] + c[...] + e[...]) * jnp.float32(1.0 / 3.0)

    return pl.pallas_call(
        body,
        out_shape=jax.ShapeDtypeStruct((n_items, d), jnp.float32),
        grid=(10,),
        in_specs=[pl.BlockSpec((blk, d), lambda i: (i, 0))] * 4,
        out_specs=pl.BlockSpec((blk, d), lambda i: (i, 0)),
    )(t0, t1, p2a, p2b)


def kernel(item_embedding, cate_embedding, edge_index, edge_vals):
    n_items = item_embedding.shape[0]
    t0 = jnp.concatenate([item_embedding, cate_embedding], axis=0)
    n_total = t0.shape[0]
    n_edges = edge_index.shape[1]

    unit = _NS * 96
    n_pad = ((n_total + unit - 1) // unit) * unit
    t0p = jnp.pad(t0, ((0, n_pad - n_total), (0, 0)))

    epw = n_edges // _NW
    K = 80
    nb = epw // K
    src_b = edge_index[0].astype(jnp.int32).reshape(_NW, nb, K)
    dst_b = edge_index[1].astype(jnp.int32).reshape(_NW, nb, K)
    rec = jnp.concatenate([src_b, dst_b], axis=2)
    rec = jnp.pad(rec, ((0, 0), (0, 2), (0, 0))).reshape(-1)
    valr = edge_vals.astype(jnp.float32).reshape(_NW, nb, K)
    valr = jnp.pad(valr, ((0, 0), (0, 2), (0, 0))).reshape(-1)

    sc_layer = _make_sc_layer(n_pad, n_edges)
    p1 = sc_layer(t0p, rec, valr)
    t1 = _tc_add(p1[0], p1[1])
    p2 = sc_layer(t1, rec, valr)
    return _tc_final(t0p[:n_items], t1[:n_items], p2[0, :n_items],
                     p2[1, :n_items], n_items)
